# baseline (device time: 26417 ns/iter reference)
import jax
import jax.numpy as jnp
from jax import lax
from jax.experimental import pallas as pl
from jax.experimental.pallas import tpu as pltpu

BM = 512


def kernel(x, dy, gamma):
    m, d = x.shape
    grid = m // BM

    def body(x_ref, dy_ref, out_ref, acc_ref, comm_ref, send_sem, recv_sem):
        step = pl.program_id(0)

        xb = x_ref[...]
        dyb = dy_ref[...]
        mu = jnp.mean(xb, axis=1, keepdims=True)
        xc = xb - mu
        var = jnp.mean(xc * xc, axis=1, keepdims=True)
        rstd = lax.rsqrt(var + 1e-5)
        xhat = xc * rstd
        pg = jnp.sum(dyb * xhat, axis=0)
        pb = jnp.sum(dyb, axis=0)

        @pl.when(step == 0)
        def _():
            acc_ref[0, :] = pg
            acc_ref[1, :] = pb

        @pl.when(step > 0)
        def _():
            acc_ref[0, :] += pg
            acc_ref[1, :] += pb

        @pl.when(step == grid - 1)
        def _():
            my_x = lax.axis_index("x")
            my_y = lax.axis_index("y")
            nbr = (my_x, 1 - my_y)

            barrier = pltpu.get_barrier_semaphore()
            pl.semaphore_signal(
                barrier, inc=1, device_id=nbr,
                device_id_type=pl.DeviceIdType.MESH,
            )
            pl.semaphore_wait(barrier, 1)

            rdma = pltpu.make_async_remote_copy(
                src_ref=acc_ref,
                dst_ref=comm_ref,
                send_sem=send_sem,
                recv_sem=recv_sem,
                device_id=nbr,
                device_id_type=pl.DeviceIdType.MESH,
            )
            rdma.start()
            rdma.wait()

            out_ref[...] = acc_ref[...] + comm_ref[...]

    return pl.pallas_call(
        body,
        grid=(grid,),
        out_shape=jax.ShapeDtypeStruct((2, d), jnp.float32),
        in_specs=[
            pl.BlockSpec((BM, d), lambda i: (i, 0)),
            pl.BlockSpec((BM, d), lambda i: (i, 0)),
        ],
        out_specs=pl.BlockSpec((2, d), lambda i: (0, 0)),
        scratch_shapes=[
            pltpu.VMEM((2, d), jnp.float32),
            pltpu.VMEM((2, d), jnp.float32),
            pltpu.SemaphoreType.DMA,
            pltpu.SemaphoreType.DMA,
        ],
        compiler_params=pltpu.CompilerParams(
            dimension_semantics=("arbitrary",),
            collective_id=0,
        ),
    )(x, dy)


# device time: 18135 ns/iter; 1.4567x vs baseline; 1.4567x over previous
import jax
import jax.numpy as jnp
from jax import lax
from jax.experimental import pallas as pl
from jax.experimental.pallas import tpu as pltpu

BM = 512


def kernel(x, dy, gamma):
    m, d = x.shape
    half = m // 2
    grid = half // BM

    my_x_outer = lax.axis_index("x")
    off = jnp.full((1,), my_x_outer * grid, dtype=jnp.int32)

    def body(off_ref, x_ref, dy_ref, out_ref, acc_ref, comm_ref,
             send_sems, recv_sems):
        step = pl.program_id(0)

        xb = x_ref[...]
        dyb = dy_ref[...]
        mu = jnp.mean(xb, axis=1, keepdims=True)
        xc = xb - mu
        var = jnp.mean(xc * xc, axis=1, keepdims=True)
        rstd = lax.rsqrt(var + 1e-5)
        xhat = xc * rstd
        pg = jnp.sum(dyb * xhat, axis=0)
        pb = jnp.sum(dyb, axis=0)

        @pl.when(step == 0)
        def _():
            acc_ref[0, :] = pg
            acc_ref[1, :] = pb

        @pl.when(step > 0)
        def _():
            acc_ref[0, :] += pg
            acc_ref[1, :] += pb

        @pl.when(step == grid - 1)
        def _():
            my_x = lax.axis_index("x")
            my_y = lax.axis_index("y")
            peers = [
                (1 - my_x, my_y),
                (my_x, 1 - my_y),
                (1 - my_x, 1 - my_y),
            ]

            barrier = pltpu.get_barrier_semaphore()
            for p in peers:
                pl.semaphore_signal(
                    barrier, inc=1, device_id=p,
                    device_id_type=pl.DeviceIdType.MESH,
                )
            pl.semaphore_wait(barrier, 3)

            rdmas = []
            for k, p in enumerate(peers):
                rdma = pltpu.make_async_remote_copy(
                    src_ref=acc_ref,
                    dst_ref=comm_ref.at[k],
                    send_sem=send_sems.at[k],
                    recv_sem=recv_sems.at[k],
                    device_id=p,
                    device_id_type=pl.DeviceIdType.MESH,
                )
                rdma.start()
                rdmas.append(rdma)
            for rdma in rdmas:
                rdma.wait()

            out_ref[...] = (
                acc_ref[...]
                + comm_ref[0, :, :]
                + comm_ref[1, :, :]
                + comm_ref[2, :, :]
            )

    grid_spec = pltpu.PrefetchScalarGridSpec(
        num_scalar_prefetch=1,
        grid=(grid,),
        in_specs=[
            pl.BlockSpec((BM, d), lambda i, off_ref: (off_ref[0] + i, 0)),
            pl.BlockSpec((BM, d), lambda i, off_ref: (off_ref[0] + i, 0)),
        ],
        out_specs=pl.BlockSpec((2, d), lambda i, off_ref: (0, 0)),
        scratch_shapes=[
            pltpu.VMEM((2, d), jnp.float32),
            pltpu.VMEM((3, 2, d), jnp.float32),
            pltpu.SemaphoreType.DMA((3,)),
            pltpu.SemaphoreType.DMA((3,)),
        ],
    )

    return pl.pallas_call(
        body,
        grid_spec=grid_spec,
        out_shape=jax.ShapeDtypeStruct((2, d), jnp.float32),
        compiler_params=pltpu.CompilerParams(
            dimension_semantics=("arbitrary",),
            collective_id=0,
        ),
    )(off, x, dy)
